# Initial kernel scaffold; baseline (speedup 1.0000x reference)
import jax
import jax.numpy as jnp
from jax.experimental import pallas as pl


def kernel(x, labels, W, b):
    raise NotImplementedError("write your pallas kernel here")


# speedup vs baseline: 8.9809x; 8.9809x over previous
import functools

import jax
import jax.numpy as jnp
from jax.experimental import pallas as pl
from jax.experimental.pallas import tpu as pltpu

NEG_INF = -1e30


def _fused_loss_kernel(x_ref, labels_ref, w_ref, b_ref, out_ref,
                       m_ref, s_ref, g_ref, *, tile_c, n_classes, n_pos):
    c = pl.program_id(0)
    nc = pl.num_programs(0)
    rows = x_ref.shape[0]

    @pl.when(c == 0)
    def _init():
        m_ref[...] = jnp.full_like(m_ref, NEG_INF)
        s_ref[...] = jnp.zeros_like(s_ref)
        g_ref[...] = jnp.zeros_like(g_ref)

    logits = jnp.dot(x_ref[...], w_ref[...],
                     preferred_element_type=jnp.float32) + b_ref[...]

    col0 = c * tile_c
    iota = jax.lax.broadcasted_iota(jnp.int32, (rows, tile_c), 1)
    valid = (col0 + iota) < n_classes
    logits = jnp.where(valid, logits, NEG_INF)

    tile_max = jnp.max(logits, axis=1, keepdims=True)
    m_old = m_ref[...]
    m_new = jnp.maximum(m_old, tile_max)
    s_ref[...] = (s_ref[...] * jnp.exp(m_old - m_new)
                  + jnp.sum(jnp.exp(logits - m_new), axis=1, keepdims=True))
    m_ref[...] = m_new

    for j in range(n_pos):
        idx = labels_ref[:, j:j + 1] - col0
        onehot = iota == idx
        g_ref[:, j:j + 1] += jnp.sum(jnp.where(onehot, logits, 0.0),
                                     axis=1, keepdims=True)

    @pl.when(c == nc - 1)
    def _finish():
        m = m_ref[...]
        s = s_ref[...]
        g = g_ref[...]
        lab = labels_ref[...]
        e = jnp.exp(g - m)
        sum_distinct = e[:, 0:1]
        for j in range(1, n_pos):
            dup = (lab[:, j:j + 1] == lab[:, :j]).any(axis=1, keepdims=True)
            sum_distinct = sum_distinct + jnp.where(dup, 0.0, e[:, j:j + 1])
        z_excl = s - sum_distinct + e
        lse = m + jnp.log(z_excl)
        total = jnp.sum(lse - g) / (lse.shape[0] * lse.shape[1])
        out_ref[...] = total.reshape(1, 1)


@functools.partial(jax.jit, static_argnames=())
def kernel(x, labels, W, b):
    B, D = x.shape
    C = W.shape[1]
    P = labels.shape[1]
    TILE_C = 512
    grid = (pl.cdiv(C, TILE_C),)

    b2 = b.reshape(1, C)

    out = pl.pallas_call(
        functools.partial(_fused_loss_kernel, tile_c=TILE_C, n_classes=C,
                          n_pos=P),
        grid=grid,
        in_specs=[
            pl.BlockSpec((B, D), lambda c: (0, 0)),
            pl.BlockSpec((B, P), lambda c: (0, 0)),
            pl.BlockSpec((D, TILE_C), lambda c: (0, c)),
            pl.BlockSpec((1, TILE_C), lambda c: (0, c)),
        ],
        out_specs=pl.BlockSpec((1, 1), lambda c: (0, 0)),
        out_shape=jax.ShapeDtypeStruct((1, 1), jnp.float32),
        scratch_shapes=[
            pltpu.VMEM((B, 1), jnp.float32),
            pltpu.VMEM((B, 1), jnp.float32),
            pltpu.VMEM((B, P), jnp.float32),
        ],
        compiler_params=pltpu.CompilerParams(
            dimension_semantics=("arbitrary",),
        ),
    )(x, labels, W, b2)
    return out[0, 0]
